# Initial kernel scaffold; baseline (speedup 1.0000x reference)
import functools

import jax
import jax.numpy as jnp
from jax.experimental import pallas as pl
from jax.experimental.pallas import tpu as pltpu

N = 1024
H = 8
C = 16
FEAT = H * C
TJ = 256
NEG = -1e30


def _proj_kernel(x_ref, wl_ref, bl_ref, wr_ref, br_ref,
                 xl_ref, xlt_ref, xr_ref):
    x = x_ref[...]
    xl = jnp.dot(x, wl_ref[...], preferred_element_type=jnp.float32) + bl_ref[...]
    xr = jnp.dot(x, wr_ref[...], preferred_element_type=jnp.float32) + br_ref[...]
    xl_ref[...] = xl
    xlt_ref[...] = xl.T
    xr_ref[...] = xr


def _attn_kernel(xl_ref, xlt_ref, xr_ref, adj_ref, att_ref, bias_ref,
                 out_ref, *, apply_elu):
    j0 = pl.program_id(0) * TJ
    adj_t = adj_ref[...].T
    row_j = jax.lax.broadcasted_iota(jnp.int32, (TJ, N), 0) + j0
    col_i = jax.lax.broadcasted_iota(jnp.int32, (TJ, N), 1)
    diag = row_j == col_i
    valid = jnp.where(diag, True, adj_t != 0)

    outs = []
    for h in range(H):
        acc = jnp.zeros((TJ, N), jnp.float32)
        for c in range(C):
            f = h * C + c
            a = att_ref[h, c]
            s = xr_ref[:, f:f + 1] + xlt_ref[f:f + 1, :]
            acc = acc + a * jnp.maximum(s, 0.2 * s)
        acc = jnp.where(valid, acc, NEG)
        m = jnp.max(acc, axis=1, keepdims=True)
        p = jnp.exp(acc - m)
        den = jnp.sum(p, axis=1, keepdims=True) + 1e-16
        alpha = p / den
        agg = jnp.dot(alpha, xl_ref[:, h * C:(h + 1) * C],
                      preferred_element_type=jnp.float32)
        outs.append(agg)
    out = jnp.concatenate(outs, axis=1) + bias_ref[...]
    if apply_elu:
        out = jnp.where(out > 0, out, jnp.exp(jnp.minimum(out, 0.0)) - 1.0)
    out_ref[...] = out


def _project(x, wl, bl, wr, br):
    return pl.pallas_call(
        _proj_kernel,
        out_shape=[
            jax.ShapeDtypeStruct((N, FEAT), jnp.float32),
            jax.ShapeDtypeStruct((FEAT, N), jnp.float32),
            jax.ShapeDtypeStruct((N, FEAT), jnp.float32),
        ],
    )(x, wl, bl.reshape(1, FEAT), wr, br.reshape(1, FEAT))


def _attention(xl, xlt, xr, adj, att, bias, apply_elu):
    grid = (N // TJ,)
    return pl.pallas_call(
        functools.partial(_attn_kernel, apply_elu=apply_elu),
        grid=grid,
        in_specs=[
            pl.BlockSpec((N, FEAT), lambda j: (0, 0)),
            pl.BlockSpec((FEAT, N), lambda j: (0, 0)),
            pl.BlockSpec((TJ, FEAT), lambda j: (j, 0)),
            pl.BlockSpec((N, TJ), lambda j: (0, j)),
            pl.BlockSpec(memory_space=pltpu.SMEM),
            pl.BlockSpec((1, FEAT), lambda j: (0, 0)),
        ],
        out_specs=pl.BlockSpec((TJ, FEAT), lambda j: (j, 0)),
        out_shape=jax.ShapeDtypeStruct((N, FEAT), jnp.float32),
        compiler_params=pltpu.CompilerParams(
            dimension_semantics=("parallel",)),
    )(xl, xlt, xr, adj, att, bias.reshape(1, FEAT))


def kernel(input, adj, Wl1, bl1, Wr1, br1, att1, bias1,
           Wl2, bl2, Wr2, br2, att2, bias2):
    b, n, ic, nf = input.shape
    x = input.reshape(n, ic * nf)
    adj32 = adj.astype(jnp.int32)
    xl1, xlt1, xr1 = _project(x, Wl1, bl1, Wr1, br1)
    h1 = _attention(xl1, xlt1, xr1, adj32, att1, bias1, apply_elu=False)
    xl2, xlt2, xr2 = _project(h1, Wl2, bl2, Wr2, br2)
    h2 = _attention(xl2, xlt2, xr2, adj32, att2, bias2, apply_elu=True)
    return h2.reshape(b, n, H * C)


# speedup vs baseline: 868.8808x; 868.8808x over previous
import functools

import jax
import jax.numpy as jnp
from jax.experimental import pallas as pl
from jax.experimental.pallas import tpu as pltpu

N = 1024
H = 8
C = 16
FEAT = H * C
TJ = 256
NEG = -1e30


def _proj_kernel(x_ref, wl_ref, bl_ref, wr_ref, br_ref,
                 xl_ref, xlt_ref, xr_ref):
    x = x_ref[...]
    xl = jnp.dot(x, wl_ref[...], preferred_element_type=jnp.float32) + bl_ref[...]
    xr = jnp.dot(x, wr_ref[...], preferred_element_type=jnp.float32) + br_ref[...]
    xl_ref[...] = xl
    xlt_ref[...] = xl.T
    xr_ref[...] = xr


def _attn_kernel(xl_ref, xlt_ref, xr_ref, adj_ref, att_ref, bias_ref,
                 out_ref, *, apply_elu):
    j0 = pl.program_id(0) * TJ
    adj_t = adj_ref[...].T
    row_j = jax.lax.broadcasted_iota(jnp.int32, (TJ, N), 0) + j0
    col_i = jax.lax.broadcasted_iota(jnp.int32, (TJ, N), 1)
    diag = row_j == col_i
    valid = jnp.logical_or(diag, adj_t != 0)

    outs = []
    for h in range(H):
        acc = jnp.zeros((TJ, N), jnp.float32)
        for c in range(C):
            f = h * C + c
            a = att_ref[h, c]
            s = xr_ref[:, f:f + 1] + xlt_ref[f:f + 1, :]
            acc = acc + a * jnp.maximum(s, 0.2 * s)
        acc = jnp.where(valid, acc, NEG)
        m = jnp.max(acc, axis=1, keepdims=True)
        p = jnp.exp(acc - m)
        den = jnp.sum(p, axis=1, keepdims=True) + 1e-16
        alpha = p / den
        agg = jnp.dot(alpha, xl_ref[:, h * C:(h + 1) * C],
                      preferred_element_type=jnp.float32)
        outs.append(agg)
    out = jnp.concatenate(outs, axis=1) + bias_ref[...]
    if apply_elu:
        out = jnp.where(out > 0, out, jnp.exp(jnp.minimum(out, 0.0)) - 1.0)
    out_ref[...] = out


def _project(x, wl, bl, wr, br):
    return pl.pallas_call(
        _proj_kernel,
        out_shape=[
            jax.ShapeDtypeStruct((N, FEAT), jnp.float32),
            jax.ShapeDtypeStruct((FEAT, N), jnp.float32),
            jax.ShapeDtypeStruct((N, FEAT), jnp.float32),
        ],
    )(x, wl, bl.reshape(1, FEAT), wr, br.reshape(1, FEAT))


def _attention(xl, xlt, xr, adj, att, bias, apply_elu):
    grid = (N // TJ,)
    return pl.pallas_call(
        functools.partial(_attn_kernel, apply_elu=apply_elu),
        grid=grid,
        in_specs=[
            pl.BlockSpec((N, FEAT), lambda j: (0, 0)),
            pl.BlockSpec((FEAT, N), lambda j: (0, 0)),
            pl.BlockSpec((TJ, FEAT), lambda j: (j, 0)),
            pl.BlockSpec((N, TJ), lambda j: (0, j)),
            pl.BlockSpec(memory_space=pltpu.SMEM),
            pl.BlockSpec((1, FEAT), lambda j: (0, 0)),
        ],
        out_specs=pl.BlockSpec((TJ, FEAT), lambda j: (j, 0)),
        out_shape=jax.ShapeDtypeStruct((N, FEAT), jnp.float32),
        compiler_params=pltpu.CompilerParams(
            dimension_semantics=("parallel",)),
    )(xl, xlt, xr, adj, att, bias.reshape(1, FEAT))


def kernel(input, adj, Wl1, bl1, Wr1, br1, att1, bias1,
           Wl2, bl2, Wr2, br2, att2, bias2):
    b, n, ic, nf = input.shape
    x = input.reshape(n, ic * nf)
    adj32 = adj.astype(jnp.int32)
    xl1, xlt1, xr1 = _project(x, Wl1, bl1, Wr1, br1)
    h1 = _attention(xl1, xlt1, xr1, adj32, att1, bias1, apply_elu=False)
    xl2, xlt2, xr2 = _project(h1, Wl2, bl2, Wr2, br2)
    h2 = _attention(xl2, xlt2, xr2, adj32, att2, bias2, apply_elu=True)
    return h2.reshape(b, n, H * C)
